# Initial kernel scaffold; baseline (speedup 1.0000x reference)
import jax
import jax.numpy as jnp
from jax.experimental import pallas as pl


def kernel(attention_scores, target_indices, video_length):
    raise NotImplementedError("write your pallas kernel here")


# speedup vs baseline: 2.4080x; 2.4080x over previous
import functools

import jax
import jax.numpy as jnp
from jax import lax
from jax.experimental import pallas as pl
from jax.experimental.pallas import tpu as pltpu
from jax.experimental.pallas import tpu_sc as plsc

L_PLANES = 16
B = 8
T_B = 512
A = 512
T_BLK = 128
N_WORKERS = 32
WINDOW = 15
SENTINEL = 1024


def _sc_body(ti_hbm, vl_hbm, aidx_hbm, sumt_hbm, cam_v, vl_v, out_v, sumt_v):
    wid = lax.axis_index("s") * 2 + lax.axis_index("c")
    b = wid // 4
    t0 = (wid % 4) * T_BLK

    pltpu.sync_copy(ti_hbm.at[b], cam_v)
    pltpu.sync_copy(vl_hbm, vl_v)

    lane = lax.iota(jnp.int32, 16)
    t_vec = plsc.load_gather(vl_v, [jnp.full((16,), b, jnp.int32)])

    def row(r, _):
        t = t0 + r
        start = jnp.minimum(2 * t_vec - 16, jnp.maximum(0, 2 * t - 7))
        idx = start + lane
        idx = jnp.where(idx < 0, idx + 1024, idx)
        idx = jnp.clip(idx, 0, 1023)
        a = plsc.load_gather(cam_v, [idx])
        valid = (t < t_vec) & (lane < WINDOW)
        a = jnp.where(valid, a, SENTINEL)
        out_v[pl.ds(r * 16, 16)] = a
        return _

    lax.fori_loop(0, T_BLK, row, None)
    pltpu.sync_copy(out_v, aidx_hbm.at[b, pl.ds(t0 * 16, T_BLK * 16)])

    @pl.when(wid == 0)
    def _():
        idx8 = jnp.where(lane < B, lane, 0)
        vals = plsc.load_gather(vl_v, [idx8])
        vals = jnp.where(lane < B, vals, 0)
        s = jnp.sum(vals)
        sumt_v[...] = jnp.full((16,), s, jnp.int32)
        pltpu.sync_copy(sumt_v, sumt_hbm)


def _sc_gather(ti, vl16):
    fn = functools.partial(
        pl.kernel,
        mesh=plsc.VectorSubcoreMesh(
            core_axis_name="c", subcore_axis_name="s", num_cores=2
        ),
        compiler_params=pltpu.CompilerParams(needs_layout_passes=False),
        out_type=(
            jax.ShapeDtypeStruct((B, T_B * 16), jnp.int32),
            jax.ShapeDtypeStruct((16,), jnp.int32),
        ),
        scratch_types=[
            pltpu.VMEM((1024,), jnp.int32),
            pltpu.VMEM((16,), jnp.int32),
            pltpu.VMEM((T_BLK * 16,), jnp.int32),
            pltpu.VMEM((16,), jnp.int32),
        ],
    )(_sc_body)
    return fn(ti, vl16)


def _tc_body(aidx_ref, att_ref, out_ref, acc_ref):
    b = pl.program_id(0)
    tb = pl.program_id(1)
    k = pl.program_id(2)

    @pl.when((b == 0) & (tb == 0) & (k == 0))
    def _():
        out_ref[...] = jnp.zeros_like(out_ref)

    @pl.when(k == 0)
    def _():
        acc_ref[...] = jnp.zeros_like(acc_ref)

    acc_ref[...] += att_ref[0, 0, 0]

    @pl.when(k == L_PLANES - 1)
    def _():
        logp = jnp.log(acc_ref[...] * (1.0 / L_PLANES) + 1e-8)
        cols = lax.broadcasted_iota(jnp.int32, (T_BLK, A), 1)
        counts = jnp.zeros((T_BLK, A), jnp.float32)
        for j in range(WINDOW):
            aj = aidx_ref[0, :, j : j + 1]
            counts += jnp.where(aj == cols, 1.0, 0.0)
        out_ref[...] = out_ref[...] + jnp.sum(counts * logp)


def _tc_loss(aidx, att):
    return pl.pallas_call(
        _tc_body,
        grid=(B, T_B // T_BLK, L_PLANES),
        in_specs=[
            pl.BlockSpec((1, T_BLK, 16), lambda b, tb, k: (b, tb, 0)),
            pl.BlockSpec(
                (1, 1, 1, T_BLK, A),
                lambda b, tb, k: (k // 8, b, k % 8, tb, 0),
            ),
        ],
        out_specs=pl.BlockSpec((1, 1), lambda b, tb, k: (0, 0)),
        out_shape=jax.ShapeDtypeStruct((1, 1), jnp.float32),
        scratch_shapes=[pltpu.VMEM((T_BLK, A), jnp.float32)],
    )(aidx, att)


def kernel(attention_scores, target_indices, video_length):
    ti = target_indices.astype(jnp.int32)
    vl16 = jnp.zeros((16,), jnp.int32).at[:B].set(video_length.astype(jnp.int32))

    aidx_flat, sumt = _sc_gather(ti, vl16)
    aidx = aidx_flat.reshape(B, T_B, 16)

    num = _tc_loss(aidx, attention_scores)[0, 0]
    return -num / sumt[0].astype(jnp.float32)


# speedup vs baseline: 9.8673x; 4.0976x over previous
import functools

import jax
import jax.numpy as jnp
from jax import lax
from jax.experimental import pallas as pl
from jax.experimental.pallas import tpu as pltpu
from jax.experimental.pallas import tpu_sc as plsc

L_PLANES = 16
B = 8
T_B = 512
A = 512
T_BLK = 128
N_WORKERS = 32
WINDOW = 15
SENTINEL = 1024


def _sc_body(ti_hbm, vl_hbm, aidx_hbm, sumt_hbm, cam_v, vl_v, out_v, sumt_v):
    wid = lax.axis_index("s") * 2 + lax.axis_index("c")
    b = wid // 4
    t0 = (wid % 4) * T_BLK

    pltpu.sync_copy(ti_hbm.at[b], cam_v)
    pltpu.sync_copy(vl_hbm, vl_v)

    lane = lax.iota(jnp.int32, 16)
    t_vec = plsc.load_gather(vl_v, [jnp.full((16,), b, jnp.int32)])

    def row(r, _):
        t = t0 + r
        start = jnp.minimum(2 * t_vec - 16, jnp.maximum(0, 2 * t - 7))
        idx = start + lane
        idx = jnp.where(idx < 0, idx + 1024, idx)
        idx = jnp.clip(idx, 0, 1023)
        a = plsc.load_gather(cam_v, [idx])
        valid = (t < t_vec) & (lane < WINDOW)
        a = jnp.where(valid, a, SENTINEL)
        out_v[pl.ds(r * 16, 16)] = a
        return _

    lax.fori_loop(0, T_BLK, row, None)
    pltpu.sync_copy(out_v, aidx_hbm.at[b, pl.ds(t0 * 16, T_BLK * 16)])

    @pl.when(wid == 0)
    def _():
        idx8 = jnp.where(lane < B, lane, 0)
        vals = plsc.load_gather(vl_v, [idx8])
        vals = jnp.where(lane < B, vals, 0)
        s = jnp.sum(vals)
        sumt_v[...] = jnp.full((16,), s, jnp.int32)
        pltpu.sync_copy(sumt_v, sumt_hbm)


def _sc_gather(ti, vl16):
    fn = functools.partial(
        pl.kernel,
        mesh=plsc.VectorSubcoreMesh(
            core_axis_name="c", subcore_axis_name="s", num_cores=2
        ),
        compiler_params=pltpu.CompilerParams(needs_layout_passes=False),
        out_type=(
            jax.ShapeDtypeStruct((B, T_B * 16), jnp.int32),
            jax.ShapeDtypeStruct((16,), jnp.int32),
        ),
        scratch_types=[
            pltpu.VMEM((1024,), jnp.int32),
            pltpu.VMEM((16,), jnp.int32),
            pltpu.VMEM((T_BLK * 16,), jnp.int32),
            pltpu.VMEM((16,), jnp.int32),
        ],
    )(_sc_body)
    return fn(ti, vl16)


def _tc_body(aidx_ref, att_ref, out_ref):
    b = pl.program_id(0)
    tb = pl.program_id(1)

    @pl.when((b == 0) & (tb == 0))
    def _():
        out_ref[...] = jnp.zeros_like(out_ref)

    s = att_ref[0, 0, 0]
    for lh in range(1, L_PLANES):
        s = s + att_ref[lh // 8, 0, lh % 8]
    logp = jnp.log(s * (1.0 / L_PLANES) + 1e-8)
    cols = lax.broadcasted_iota(jnp.int32, (T_BLK, A), 1)
    counts = jnp.zeros((T_BLK, A), jnp.float32)
    for j in range(WINDOW):
        aj = aidx_ref[0, :, j : j + 1]
        counts += jnp.where(aj == cols, 1.0, 0.0)
    out_ref[...] = out_ref[...] + jnp.sum(counts * logp)


def _tc_loss(aidx, att):
    return pl.pallas_call(
        _tc_body,
        grid=(B, T_B // T_BLK),
        in_specs=[
            pl.BlockSpec((1, T_BLK, 16), lambda b, tb: (b, tb, 0)),
            pl.BlockSpec(
                (2, 1, 8, T_BLK, A),
                lambda b, tb: (0, b, 0, tb, 0),
            ),
        ],
        out_specs=pl.BlockSpec((1, 1), lambda b, tb: (0, 0)),
        out_shape=jax.ShapeDtypeStruct((1, 1), jnp.float32),
    )(aidx, att)


def kernel(attention_scores, target_indices, video_length):
    ti = target_indices.astype(jnp.int32)
    vl16 = jnp.zeros((16,), jnp.int32).at[:B].set(video_length.astype(jnp.int32))

    aidx_flat, sumt = _sc_gather(ti, vl16)
    aidx = aidx_flat.reshape(B, T_B, 16)

    num = _tc_loss(aidx, attention_scores)[0, 0]
    return -num / sumt[0].astype(jnp.float32)


# speedup vs baseline: 11.1623x; 1.1312x over previous
import functools

import jax
import jax.numpy as jnp
from jax import lax
from jax.experimental import pallas as pl
from jax.experimental.pallas import tpu as pltpu
from jax.experimental.pallas import tpu_sc as plsc

L_PLANES = 16
B = 8
T_B = 512
A = 512
T_BLK = 128
N_WORKERS = 32
WINDOW = 15
SENTINEL = 1024


def _sc_body(ti_hbm, vl_hbm, aidx_hbm, sumt_hbm, cam_v, vl_v, out_v, sumt_v):
    wid = lax.axis_index("s") * 2 + lax.axis_index("c")
    b = wid // 4
    t0 = (wid % 4) * T_BLK

    pltpu.sync_copy(ti_hbm.at[b], cam_v)
    pltpu.sync_copy(vl_hbm, vl_v)

    lane = lax.iota(jnp.int32, 16)
    t_vec = plsc.load_gather(vl_v, [jnp.full((16,), b, jnp.int32)])

    def row(r, _):
        t = t0 + r
        start = jnp.minimum(2 * t_vec - 16, jnp.maximum(0, 2 * t - 7))
        idx = start + lane
        idx = jnp.where(idx < 0, idx + 1024, idx)
        idx = jnp.clip(idx, 0, 1023)
        a = plsc.load_gather(cam_v, [idx])
        valid = (t < t_vec) & (lane < WINDOW)
        a = jnp.where(valid, a, SENTINEL)
        out_v[pl.ds(r * 16, 16)] = a
        return _

    lax.fori_loop(0, T_BLK, row, None)
    pltpu.sync_copy(out_v, aidx_hbm.at[b, pl.ds(t0 * 16, T_BLK * 16)])

    @pl.when(wid == 0)
    def _():
        idx8 = jnp.where(lane < B, lane, 0)
        vals = plsc.load_gather(vl_v, [idx8])
        vals = jnp.where(lane < B, vals, 0)
        s = jnp.sum(vals)
        sumt_v[...] = jnp.full((16,), s, jnp.int32)
        pltpu.sync_copy(sumt_v, sumt_hbm)


def _sc_gather(ti, vl16):
    fn = functools.partial(
        pl.kernel,
        mesh=plsc.VectorSubcoreMesh(
            core_axis_name="c", subcore_axis_name="s", num_cores=2
        ),
        compiler_params=pltpu.CompilerParams(needs_layout_passes=False),
        out_type=(
            jax.ShapeDtypeStruct((B, T_B * 16), jnp.int32),
            jax.ShapeDtypeStruct((16,), jnp.int32),
        ),
        scratch_types=[
            pltpu.VMEM((1024,), jnp.int32),
            pltpu.VMEM((16,), jnp.int32),
            pltpu.VMEM((T_BLK * 16,), jnp.int32),
            pltpu.VMEM((16,), jnp.int32),
        ],
    )(_sc_body)
    return fn(ti, vl16)


TC_BLK = 256


def _tc_body(aidx_ref, att_ref, out_ref):
    b = pl.program_id(0)
    tb = pl.program_id(1)

    @pl.when((b == 0) & (tb == 0))
    def _():
        out_ref[...] = jnp.zeros_like(out_ref)

    s = att_ref[0, 0, 0]
    for lh in range(1, L_PLANES):
        s = s + att_ref[lh // 8, 0, lh % 8]
    logp = jnp.log(s * (1.0 / L_PLANES) + 1e-8)
    cols = lax.broadcasted_iota(jnp.int32, (TC_BLK, A), 1)
    counts = jnp.zeros((TC_BLK, A), jnp.float32)
    for j in range(WINDOW):
        aj = aidx_ref[0, :, j : j + 1]
        counts += jnp.where(aj == cols, 1.0, 0.0)
    out_ref[...] = out_ref[...] + jnp.sum(counts * logp)


def _tc_loss(aidx, att):
    return pl.pallas_call(
        _tc_body,
        grid=(B, T_B // TC_BLK),
        in_specs=[
            pl.BlockSpec((1, TC_BLK, 16), lambda b, tb: (b, tb, 0)),
            pl.BlockSpec(
                (2, 1, 8, TC_BLK, A),
                lambda b, tb: (0, b, 0, tb, 0),
            ),
        ],
        out_specs=pl.BlockSpec((1, 1), lambda b, tb: (0, 0)),
        out_shape=jax.ShapeDtypeStruct((1, 1), jnp.float32),
    )(aidx, att)


def kernel(attention_scores, target_indices, video_length):
    ti = target_indices.astype(jnp.int32)
    vl16 = jnp.zeros((16,), jnp.int32).at[:B].set(video_length.astype(jnp.int32))

    aidx_flat, sumt = _sc_gather(ti, vl16)
    aidx = aidx_flat.reshape(B, T_B, 16)

    num = _tc_loss(aidx, attention_scores)[0, 0]
    return -num / sumt[0].astype(jnp.float32)
